# baseline (speedup 1.0000x reference)
import jax
import jax.numpy as jnp
from jax.experimental import pallas as pl
from jax.experimental.pallas import tpu as pltpu
from jax.experimental.pallas import tpu_sc as plsc

_W = 128
_NBUF = 8
_NW = 32
_PACK_BLK = 512
_UNPACK_BB = 512


def _pack_body(x_ref, o_ref):
    o_ref[...] = x_ref[...].T


def _sc_gather(w_rm, idx, flat, dim):
    chunks_per_w = flat // (_NW * _W)
    mesh = plsc.VectorSubcoreMesh(core_axis_name="c", subcore_axis_name="s")

    @pl.kernel(
        out_type=jax.ShapeDtypeStruct((flat, dim), w_rm.dtype),
        mesh=mesh,
        compiler_params=pltpu.CompilerParams(use_tc_tiling_on_sc=False),
        scratch_types=[
            pltpu.VMEM((chunks_per_w, _W), jnp.int32),
            pltpu.VMEM((_NBUF, _W, dim), jnp.float32),
            pltpu.SemaphoreType.DMA((_NBUF,)),
            pltpu.SemaphoreType.DMA((_NBUF,)),
            pltpu.SemaphoreType.DMA,
        ],
    )
    def gather_kernel(w_hbm, i_hbm, o_hbm, idx_v, rows_v, gsem, wsem, isem):
        wid = jax.lax.axis_index("s") * 2 + jax.lax.axis_index("c")
        base = wid * (chunks_per_w * _W)
        pltpu.async_copy(i_hbm.at[wid], idx_v, isem).wait()

        def start_gather(c, b):
            pltpu.async_copy(w_hbm.at[idx_v.at[c]], rows_v.at[b], gsem.at[b])

        for b in range(_NBUF):
            start_gather(b, b)

        @pl.loop(0, chunks_per_w, step=_NBUF)
        def _(g0):
            for b in range(_NBUF):
                c = g0 + b
                pltpu.make_async_copy(
                    w_hbm.at[idx_v.at[c]], rows_v.at[b], gsem.at[b]
                ).wait()
                pltpu.async_copy(
                    rows_v.at[b], o_hbm.at[pl.ds(base + c * _W, _W)], wsem.at[b]
                )
                pltpu.make_async_copy(
                    rows_v.at[b], o_hbm.at[pl.ds(base + c * _W, _W)], wsem.at[b]
                ).wait()

                @pl.when(c + _NBUF < chunks_per_w)
                def _():
                    start_gather(c + _NBUF, b)

    return gather_kernel(w_rm, idx)


def kernel(token_ids, weight):
    batch, seq = token_ids.shape
    num_rows, dim = weight.shape
    flat = batch * seq
    idx = token_ids.T.reshape(_NW, flat // (_NW * _W), _W).astype(jnp.int32)

    n_blks = pl.cdiv(num_rows, _PACK_BLK)
    pack_rows = n_blks * _PACK_BLK
    w_rm = pl.pallas_call(
        _pack_body,
        grid=(n_blks,),
        in_specs=[pl.BlockSpec((dim, _PACK_BLK), lambda c: (0, c))],
        out_specs=pl.BlockSpec((_PACK_BLK, dim), lambda c: (c, 0)),
        out_shape=jax.ShapeDtypeStruct((pack_rows, dim), weight.dtype),
    )(weight.T)

    rows = _sc_gather(w_rm, idx, flat, dim)

    def _unpack_body(x_ref, o_ref):
        o_ref[0] = x_ref[...].T

    n_bb = batch // _UNPACK_BB
    outp = pl.pallas_call(
        _unpack_body,
        grid=(n_bb, seq),
        in_specs=[
            pl.BlockSpec((_UNPACK_BB, dim), lambda b, s: (s * n_bb + b, 0))
        ],
        out_specs=pl.BlockSpec((1, dim, _UNPACK_BB), lambda b, s: (s, 0, b)),
        out_shape=jax.ShapeDtypeStruct((seq, dim, batch), weight.dtype),
    )(rows)
    return jnp.transpose(outp, (2, 0, 1))


def _debug_layouts():
    import jax as _jax
    import jax.numpy as _jnp

    try:
        tok = _jnp.zeros((16384, 50), _jnp.int32)
        w = _jnp.zeros((1000000, 32), _jnp.float32)
        txt = _jax.jit(kernel).lower(tok, w).compile().as_text()
    except Exception as e:
        print("HLODBG-skip:", repr(e)[:200])
        return
    import re
    for line in txt.splitlines():
        s = line.strip()
        if re.search(r"entry_computation_layout|copy|custom-call|fusion|bitcast", s):
            print("HLODBG:", s[:300])


_debug_layouts()


# speedup vs baseline: 2.0774x; 2.0774x over previous
import jax
import jax.numpy as jnp
from jax.experimental import pallas as pl
from jax.experimental.pallas import tpu as pltpu
from jax.experimental.pallas import tpu_sc as plsc

_W = 128
_NBUF = 8
_NW = 32
_PACK_BLK = 4096
_UNPACK_BB = 2048


def _pack_body(x_ref, o_ref):
    o_ref[...] = x_ref[...].T


def _sc_gather(w_rm, idx, flat, dim):
    chunks_per_w = flat // (_NW * _W)
    mesh = plsc.VectorSubcoreMesh(core_axis_name="c", subcore_axis_name="s")

    @pl.kernel(
        out_type=jax.ShapeDtypeStruct((flat, dim), w_rm.dtype),
        mesh=mesh,
        compiler_params=pltpu.CompilerParams(use_tc_tiling_on_sc=False),
        scratch_types=[
            pltpu.VMEM((chunks_per_w, _W), jnp.int32),
            pltpu.VMEM((_NBUF, _W, dim), jnp.float32),
            pltpu.SemaphoreType.DMA((_NBUF,)),
            pltpu.SemaphoreType.DMA((_NBUF,)),
            pltpu.SemaphoreType.DMA,
        ],
    )
    def gather_kernel(w_hbm, i_hbm, o_hbm, idx_v, rows_v, gsem, wsem, isem):
        wid = jax.lax.axis_index("s") * 2 + jax.lax.axis_index("c")
        base = wid * (chunks_per_w * _W)
        pltpu.async_copy(i_hbm.at[wid], idx_v, isem).wait()

        def start_gather(c, b):
            pltpu.async_copy(w_hbm.at[idx_v.at[c]], rows_v.at[b], gsem.at[b])

        for b in range(_NBUF):
            start_gather(b, b)

        @pl.loop(0, chunks_per_w, step=_NBUF)
        def _(g0):
            for b in range(_NBUF):
                c = g0 + b
                pltpu.make_async_copy(
                    w_hbm.at[idx_v.at[c]], rows_v.at[b], gsem.at[b]
                ).wait()
                pltpu.async_copy(
                    rows_v.at[b], o_hbm.at[pl.ds(base + c * _W, _W)], wsem.at[b]
                )
                pltpu.make_async_copy(
                    rows_v.at[b], o_hbm.at[pl.ds(base + c * _W, _W)], wsem.at[b]
                ).wait()

                @pl.when(c + _NBUF < chunks_per_w)
                def _():
                    start_gather(c + _NBUF, b)

    return gather_kernel(w_rm, idx)


def kernel(token_ids, weight):
    batch, seq = token_ids.shape
    num_rows, dim = weight.shape
    flat = batch * seq
    idx = token_ids.T.reshape(_NW, flat // (_NW * _W), _W).astype(jnp.int32)

    n_blks = pl.cdiv(num_rows, _PACK_BLK)
    pack_rows = n_blks * _PACK_BLK
    w_rm = pl.pallas_call(
        _pack_body,
        grid=(n_blks,),
        in_specs=[pl.BlockSpec((dim, _PACK_BLK), lambda c: (0, c))],
        out_specs=pl.BlockSpec((_PACK_BLK, dim), lambda c: (c, 0)),
        out_shape=jax.ShapeDtypeStruct((pack_rows, dim), weight.dtype),
    )(weight.T)

    rows = _sc_gather(w_rm, idx, flat, dim)

    def _unpack_body(x_ref, o_ref):
        o_ref[0] = x_ref[...].T

    n_bb = batch // _UNPACK_BB
    outp = pl.pallas_call(
        _unpack_body,
        grid=(n_bb, seq),
        in_specs=[
            pl.BlockSpec((_UNPACK_BB, dim), lambda b, s: (s * n_bb + b, 0))
        ],
        out_specs=pl.BlockSpec((1, dim, _UNPACK_BB), lambda b, s: (s, 0, b)),
        out_shape=jax.ShapeDtypeStruct((seq, dim, batch), weight.dtype),
    )(rows)
    return jnp.transpose(outp, (2, 0, 1))


def _debug_layouts():
    import jax as _jax
    import jax.numpy as _jnp

    try:
        tok = _jnp.zeros((16384, 50), _jnp.int32)
        w = _jnp.zeros((1000000, 32), _jnp.float32)
        txt = _jax.jit(kernel).lower(tok, w).compile().as_text()
    except Exception as e:
        print("HLODBG-skip:", repr(e)[:200])
        return
    import re
    for line in txt.splitlines():
        s = line.strip()
        if re.search(r"entry_computation_layout|copy|custom-call|fusion|bitcast", s):
            print("HLODBG:", s[:300])


_debug_layouts()


# speedup vs baseline: 2.2725x; 1.0939x over previous
import jax
import jax.numpy as jnp
from jax.experimental import pallas as pl
from jax.experimental.pallas import tpu as pltpu
from jax.experimental.pallas import tpu_sc as plsc

_W = 128
_NBUF = 4
_NW = 32
_QBITS = 18
_PACK_BLK = 2048


def _pack_body(x0_ref, x1_ref, x2_ref, x3_ref, o_ref):
    o_ref[:, 0:32] = x0_ref[...].T
    o_ref[:, 32:64] = x1_ref[...].T
    o_ref[:, 64:96] = x2_ref[...].T
    o_ref[:, 96:128] = x3_ref[...].T


def kernel(token_ids, weight):
    batch, seq = token_ids.shape
    num_rows, dim = weight.shape
    q = 1 << _QBITS
    n_win = seq * (batch // _W)
    win_per_w = n_win // _NW

    w_t = weight.T
    idx3 = token_ids.T.reshape(seq, batch // _W, _W).astype(jnp.int32)

    blks_per_q = q // _PACK_BLK
    last_blk = pl.cdiv(num_rows, _PACK_BLK) - 1
    in_spec = lambda k: pl.BlockSpec(
        (dim, _PACK_BLK),
        lambda c, _k=k: (0, jnp.minimum(_k * blks_per_q + c, last_blk)),
    )
    lines = pl.pallas_call(
        _pack_body,
        grid=(blks_per_q,),
        in_specs=[in_spec(0), in_spec(1), in_spec(2), in_spec(3)],
        out_specs=pl.BlockSpec((_PACK_BLK, 128), lambda c: (c, 0)),
        out_shape=jax.ShapeDtypeStruct((q, 128), weight.dtype),
    )(w_t, w_t, w_t, w_t)

    mesh = plsc.VectorSubcoreMesh(core_axis_name="c", subcore_axis_name="s")

    @pl.kernel(
        out_type=jax.ShapeDtypeStruct((seq, dim, batch), weight.dtype),
        mesh=mesh,
        compiler_params=pltpu.CompilerParams(needs_layout_passes=False),
        scratch_types=[
            pltpu.VMEM((_NBUF, _W), jnp.int32),
            pltpu.VMEM((_NBUF, _W), jnp.int32),
            pltpu.VMEM((_NBUF, _W, _W), jnp.float32),
            pltpu.VMEM((_NBUF, dim, _W), jnp.float32),
            pltpu.SemaphoreType.DMA((_NBUF,)),
            pltpu.SemaphoreType.DMA((_NBUF,)),
            pltpu.SemaphoreType.DMA((_NBUF,)),
        ],
    )
    def gather_kernel(l_hbm, i_hbm, o_hbm, ti_v, li_v, ln_v, ot_v,
                      isem, gsem, osem):
        wid = jax.lax.axis_index("s") * 2 + jax.lax.axis_index("c")
        base_w = wid * win_per_w
        lane = jax.lax.broadcasted_iota(jnp.int32, (16,), 0)

        def win_sb(t):
            w = base_w + t
            return w // (batch // _W), w % (batch // _W)

        def start_idx(t, b):
            s, bb = win_sb(t)
            pltpu.async_copy(i_hbm.at[s, bb], ti_v.at[b], isem.at[b])

        def start_gather(b):
            pltpu.async_copy(l_hbm.at[li_v.at[b]], ln_v.at[b], gsem.at[b])

        for b in range(_NBUF):
            start_idx(b, b)

        @pl.loop(0, win_per_w, step=_NBUF)
        def _(t0):
            for b in range(_NBUF):
                t = t0 + b
                s, bb = win_sb(t)
                pltpu.make_async_copy(
                    i_hbm.at[s, bb], ti_v.at[b], isem.at[b]
                ).wait()
                @pl.loop(0, _W, step=16)
                def _(j):
                    li_v.at[b][pl.ds(j, 16)] = (
                        ti_v.at[b][pl.ds(j, 16)] & (q - 1)
                    )
                start_gather(b)
                pltpu.make_async_copy(
                    l_hbm.at[li_v.at[b]], ln_v.at[b], gsem.at[b]
                ).wait()

                for g in range(8):
                    rv = ti_v.at[b][pl.ds(16 * g, 16)]
                    colb = jax.lax.shift_right_logical(rv, _QBITS) * dim
                    rowv = lane + (16 * g)

                    @pl.loop(0, dim)
                    def _(d):
                        vals = plsc.load_gather(
                            ln_v.at[b], [rowv, colb + d]
                        )
                        ot_v.at[b, d][pl.ds(16 * g, 16)] = vals

                for kk in range(4):
                    pltpu.async_copy(
                        ot_v.at[b, pl.ds(8 * kk, 8)],
                        o_hbm.at[s, pl.ds(8 * kk, 8), pl.ds(bb * _W, _W)],
                        osem.at[b],
                    )
                for kk in range(4):
                    pltpu.make_async_copy(
                        ot_v.at[b, pl.ds(8 * kk, 8)],
                        o_hbm.at[s, pl.ds(8 * kk, 8), pl.ds(bb * _W, _W)],
                        osem.at[b],
                    ).wait()

                @pl.when(t + _NBUF < win_per_w)
                def _():
                    start_idx(t + _NBUF, b)

    outp = gather_kernel(lines, idx3)
    return jnp.transpose(outp, (2, 0, 1))


def _debug_layouts():
    import jax as _jax
    import jax.numpy as _jnp

    try:
        tok = _jnp.zeros((16384, 50), _jnp.int32)
        w = _jnp.zeros((1000000, 32), _jnp.float32)
        txt = _jax.jit(kernel).lower(tok, w).compile().as_text()
    except Exception as e:
        print("HLODBG-skip:", repr(e)[:200])
        return
    import re
    for line in txt.splitlines():
        s = line.strip()
        if re.search(r"entry_computation_layout|copy|custom-call|fusion|reshape", s):
            print("HLODBG:", s[:300])


_debug_layouts()


# speedup vs baseline: 2.6605x; 1.1707x over previous
import jax
import jax.numpy as jnp
from jax.experimental import pallas as pl
from jax.experimental.pallas import tpu as pltpu
from jax.experimental.pallas import tpu_sc as plsc

_W = 128
_NBUF = 4
_NW = 32
_QBITS = 18
_PACK_BLK = 4096


def _pack_body(x0_ref, x1_ref, x2_ref, x3_ref, o_ref):
    o_ref[:, 0:32] = x0_ref[...].T
    o_ref[:, 32:64] = x1_ref[...].T
    o_ref[:, 64:96] = x2_ref[...].T
    o_ref[:, 96:128] = x3_ref[...].T


def kernel(token_ids, weight):
    batch, seq = token_ids.shape
    num_rows, dim = weight.shape
    q = 1 << _QBITS
    n_win = seq * (batch // _W)
    win_per_w = n_win // _NW

    w_t = weight.T
    idx3 = token_ids.T.reshape(seq, batch // _W, _W).astype(jnp.int32)

    blks_per_q = q // _PACK_BLK
    last_blk = pl.cdiv(num_rows, _PACK_BLK) - 1
    in_spec = lambda k: pl.BlockSpec(
        (dim, _PACK_BLK),
        lambda c, _k=k: (0, jnp.minimum(_k * blks_per_q + c, last_blk)),
    )
    lines = pl.pallas_call(
        _pack_body,
        grid=(blks_per_q,),
        in_specs=[in_spec(0), in_spec(1), in_spec(2), in_spec(3)],
        out_specs=pl.BlockSpec((_PACK_BLK, 128), lambda c: (c, 0)),
        out_shape=jax.ShapeDtypeStruct((q, 128), weight.dtype),
    )(w_t, w_t, w_t, w_t)

    mesh = plsc.VectorSubcoreMesh(core_axis_name="c", subcore_axis_name="s")

    @pl.kernel(
        out_type=jax.ShapeDtypeStruct((seq, dim, batch), weight.dtype),
        mesh=mesh,
        compiler_params=pltpu.CompilerParams(needs_layout_passes=False),
        scratch_types=[
            pltpu.VMEM((_NBUF, _W), jnp.int32),
            pltpu.VMEM((_NBUF, _W), jnp.int32),
            pltpu.VMEM((_NBUF, _W, _W), jnp.float32),
            pltpu.VMEM((_NBUF, dim, _W), jnp.float32),
            pltpu.SemaphoreType.DMA((_NBUF,)),
            pltpu.SemaphoreType.DMA((_NBUF,)),
            pltpu.SemaphoreType.DMA((_NBUF,)),
        ],
    )
    def gather_kernel(l_hbm, i_hbm, o_hbm, ti_v, li_v, ln_v, ot_v,
                      isem, gsem, osem):
        wid = jax.lax.axis_index("s") * 2 + jax.lax.axis_index("c")
        base_w = wid * win_per_w
        lane = jax.lax.broadcasted_iota(jnp.int32, (16,), 0)

        def win_sb(t):
            w = base_w + t
            return w // (batch // _W), w % (batch // _W)

        def start_idx(t, b):
            s, bb = win_sb(t)
            pltpu.async_copy(i_hbm.at[s, bb], ti_v.at[b], isem.at[b])

        def start_gather(b):
            pltpu.async_copy(l_hbm.at[li_v.at[b]], ln_v.at[b], gsem.at[b])

        def wait_idx(b):
            pltpu.make_async_copy(
                i_hbm.at[0, 0], ti_v.at[b], isem.at[b]
            ).wait()

        def prep_gather(b):
            for j in range(0, _W, 16):
                li_v.at[b][pl.ds(j, 16)] = ti_v.at[b][pl.ds(j, 16)] & (q - 1)
            start_gather(b)

        for b in range(_NBUF):
            start_idx(b, b)
        for b in range(_NBUF - 1):
            wait_idx(b)
            prep_gather(b)

        @pl.loop(0, win_per_w, step=_NBUF)
        def _(t0):
            for b in range(_NBUF):
                t = t0 + b
                bn = (b - 1) % _NBUF
                tn = t + _NBUF - 1

                @pl.when(tn < win_per_w)
                def _():
                    wait_idx(bn)
                    prep_gather(bn)

                s, bb = win_sb(t)
                pltpu.make_async_copy(
                    l_hbm.at[li_v.at[b]], ln_v.at[b], gsem.at[b]
                ).wait()

                for g in range(8):
                    rv = ti_v.at[b][pl.ds(16 * g, 16)]
                    colb = jax.lax.shift_right_logical(rv, _QBITS) * dim
                    rowv = lane + (16 * g)
                    for d in range(dim):
                        vals = plsc.load_gather(
                            ln_v.at[b], [rowv, colb + d]
                        )
                        ot_v.at[b, d][pl.ds(16 * g, 16)] = vals

                for kk in range(4):
                    pltpu.async_copy(
                        ot_v.at[b, pl.ds(8 * kk, 8)],
                        o_hbm.at[s, pl.ds(8 * kk, 8), pl.ds(bb * _W, _W)],
                        osem.at[b],
                    )
                for kk in range(4):
                    pltpu.make_async_copy(
                        ot_v.at[b, pl.ds(8 * kk, 8)],
                        o_hbm.at[s, pl.ds(8 * kk, 8), pl.ds(bb * _W, _W)],
                        osem.at[b],
                    ).wait()

                @pl.when(t + _NBUF < win_per_w)
                def _():
                    start_idx(t + _NBUF, b)

    outp = gather_kernel(lines, idx3)
    return jnp.transpose(outp, (2, 0, 1))


def _debug_layouts():
    import jax as _jax
    import jax.numpy as _jnp

    try:
        tok = _jnp.zeros((16384, 50), _jnp.int32)
        w = _jnp.zeros((1000000, 32), _jnp.float32)
        txt = _jax.jit(kernel).lower(tok, w).compile().as_text()
    except Exception as e:
        print("HLODBG-skip:", repr(e)[:200])
        return
    import re
    for line in txt.splitlines():
        s = line.strip()
        if re.search(r"entry_computation_layout|copy|custom-call|fusion|reshape", s):
            print("HLODBG:", s[:300])


_debug_layouts()


# speedup vs baseline: 2.6620x; 1.0005x over previous
import jax
import jax.numpy as jnp
from jax.experimental import pallas as pl
from jax.experimental.pallas import tpu as pltpu
from jax.experimental.pallas import tpu_sc as plsc

_W = 128
_NBUF = 5
_NW = 32
_QBITS = 18
_PACK_BLK = 4096


def _pack_body(x0_ref, x1_ref, x2_ref, x3_ref, o_ref):
    o_ref[:, 0:32] = x0_ref[...].T
    o_ref[:, 32:64] = x1_ref[...].T
    o_ref[:, 64:96] = x2_ref[...].T
    o_ref[:, 96:128] = x3_ref[...].T


def kernel(token_ids, weight):
    batch, seq = token_ids.shape
    num_rows, dim = weight.shape
    q = 1 << _QBITS
    n_win = seq * (batch // _W)
    win_per_w = n_win // _NW

    w_t = weight.T
    idx3 = token_ids.T.reshape(seq, batch // _W, _W).astype(jnp.int32)

    blks_per_q = q // _PACK_BLK
    last_blk = pl.cdiv(num_rows, _PACK_BLK) - 1
    in_spec = lambda k: pl.BlockSpec(
        (dim, _PACK_BLK),
        lambda c, _k=k: (0, jnp.minimum(_k * blks_per_q + c, last_blk)),
    )
    lines = pl.pallas_call(
        _pack_body,
        grid=(blks_per_q,),
        in_specs=[in_spec(0), in_spec(1), in_spec(2), in_spec(3)],
        out_specs=pl.BlockSpec((_PACK_BLK, 128), lambda c: (c, 0)),
        out_shape=jax.ShapeDtypeStruct((q, 128), weight.dtype),
    )(w_t, w_t, w_t, w_t)

    mesh = plsc.VectorSubcoreMesh(core_axis_name="c", subcore_axis_name="s")

    @pl.kernel(
        out_type=jax.ShapeDtypeStruct((seq, dim, batch), weight.dtype),
        mesh=mesh,
        compiler_params=pltpu.CompilerParams(needs_layout_passes=False),
        scratch_types=[
            pltpu.VMEM((_NBUF, _W), jnp.int32),
            pltpu.VMEM((_NBUF, _W), jnp.int32),
            pltpu.VMEM((_NBUF, _W, _W), jnp.float32),
            pltpu.VMEM((_NBUF, dim, _W), jnp.float32),
            pltpu.SemaphoreType.DMA((_NBUF,)),
            pltpu.SemaphoreType.DMA((_NBUF,)),
            pltpu.SemaphoreType.DMA((_NBUF,)),
        ],
    )
    def gather_kernel(l_hbm, i_hbm, o_hbm, ti_v, li_v, ln_v, ot_v,
                      isem, gsem, osem):
        wid = jax.lax.axis_index("s") * 2 + jax.lax.axis_index("c")
        base_w = wid * win_per_w
        lane = jax.lax.broadcasted_iota(jnp.int32, (16,), 0)

        def win_sb(t):
            w = base_w + t
            return w // (batch // _W), w % (batch // _W)

        def start_idx(t, b):
            s, bb = win_sb(t)
            pltpu.async_copy(i_hbm.at[s, bb], ti_v.at[b], isem.at[b])

        def start_gather(b):
            pltpu.async_copy(l_hbm.at[li_v.at[b]], ln_v.at[b], gsem.at[b])

        def wait_idx(b):
            pltpu.make_async_copy(
                i_hbm.at[0, 0], ti_v.at[b], isem.at[b]
            ).wait()

        def prep_gather(b):
            for j in range(0, _W, 16):
                li_v.at[b][pl.ds(j, 16)] = ti_v.at[b][pl.ds(j, 16)] & (q - 1)
            start_gather(b)

        for b in range(_NBUF):
            start_idx(b, b)
        for b in range(_NBUF - 1):
            wait_idx(b)
            prep_gather(b)

        @pl.loop(0, win_per_w, step=_NBUF)
        def _(t0):
            for b in range(_NBUF):
                t = t0 + b
                bn = (b - 1) % _NBUF
                tn = t + _NBUF - 1

                @pl.when(tn < win_per_w)
                def _():
                    wait_idx(bn)
                    prep_gather(bn)

                s, bb = win_sb(t)
                pltpu.make_async_copy(
                    l_hbm.at[li_v.at[b]], ln_v.at[b], gsem.at[b]
                ).wait()

                for g in range(8):
                    rv = ti_v.at[b][pl.ds(16 * g, 16)]
                    colb = jax.lax.shift_right_logical(rv, _QBITS) * dim
                    rowv = lane + (16 * g)
                    for d in range(dim):
                        vals = plsc.load_gather(
                            ln_v.at[b], [rowv, colb + d]
                        )
                        ot_v.at[b, d][pl.ds(16 * g, 16)] = vals

                for kk in range(4):
                    pltpu.async_copy(
                        ot_v.at[b, pl.ds(8 * kk, 8)],
                        o_hbm.at[s, pl.ds(8 * kk, 8), pl.ds(bb * _W, _W)],
                        osem.at[b],
                    )
                for kk in range(4):
                    pltpu.make_async_copy(
                        ot_v.at[b, pl.ds(8 * kk, 8)],
                        o_hbm.at[s, pl.ds(8 * kk, 8), pl.ds(bb * _W, _W)],
                        osem.at[b],
                    ).wait()

                @pl.when(t + _NBUF < win_per_w)
                def _():
                    start_idx(t + _NBUF, b)

    outp = gather_kernel(lines, idx3)
    return jnp.transpose(outp, (2, 0, 1))
